# Initial kernel scaffold; baseline (speedup 1.0000x reference)
import jax
import jax.numpy as jnp
from jax.experimental import pallas as pl


def kernel(pred_rotmats, scores, grid_rotmats, medians, mads):
    raise NotImplementedError("write your pallas kernel here")


# speedup vs baseline: 1.6332x; 1.6332x over previous
import functools

import jax
import jax.numpy as jnp
from jax import lax
from jax.experimental import pallas as pl
from jax.experimental.pallas import tpu as pltpu
from jax.experimental.pallas import tpu_sc as plsc

_B = 4096
_N_SO3 = 4608
_N_PSI = 24
_N_CONES = 192

_ROW_BLK = 512
_N_ROW_BLKS = _B // _ROW_BLK

_NC = 2
_NS = 16
_NW = _NC * _NS
_PER_W = _B // _NW
_LANES = 16


def _tc_cone_body(pred_ref, grid_ref, out_ref):
    sim = lax.dot_general(
        pred_ref[...], grid_ref[...],
        dimension_numbers=(((1,), (1,)), ((), ())),
        preferred_element_type=jnp.float32,
    )
    m = jnp.max(sim, axis=1, keepdims=True)
    col = lax.broadcasted_iota(jnp.int32, sim.shape, 1)
    so3 = jnp.min(jnp.where(sim == m, col, _N_SO3), axis=1)
    out_ref[...] = so3 // _N_PSI


def _tc_cone_indices(pred9, grid9):
    return pl.pallas_call(
        _tc_cone_body,
        grid=(_N_ROW_BLKS,),
        in_specs=[
            pl.BlockSpec((_ROW_BLK, 9), lambda i: (i, 0)),
            pl.BlockSpec((_N_SO3, 9), lambda i: (0, 0)),
        ],
        out_specs=pl.BlockSpec((_ROW_BLK,), lambda i: (i,)),
        out_shape=jax.ShapeDtypeStruct((_B,), jnp.int32),
    )(pred9, grid9)


_SC_MESH = plsc.VectorSubcoreMesh(core_axis_name="c", subcore_axis_name="s")


@functools.partial(
    pl.kernel,
    mesh=_SC_MESH,
    out_type=jax.ShapeDtypeStruct((_B,), jnp.float32),
    scratch_types=[
        pltpu.VMEM((_PER_W,), jnp.int32),
        pltpu.VMEM((_PER_W,), jnp.float32),
        pltpu.VMEM((_PER_W,), jnp.float32),
        pltpu.VMEM((_PER_W,), jnp.float32),
        pltpu.VMEM((_PER_W,), jnp.float32),
        pltpu.SemaphoreType.DMA,
        pltpu.SemaphoreType.DMA,
    ],
)
def _sc_normalize(cone_hbm, scores_hbm, med_hbm, mad_hbm, out_hbm,
                  idx_v, s_v, medg_v, madg_v, o_v, sem1, sem2):
    wid = lax.axis_index("s") * _NC + lax.axis_index("c")
    base = wid * _PER_W
    pltpu.sync_copy(cone_hbm.at[pl.ds(base, _PER_W)], idx_v)
    pltpu.sync_copy(scores_hbm.at[pl.ds(base, _PER_W)], s_v)
    c1 = pltpu.async_copy(med_hbm.at[idx_v], medg_v, sem1)
    c2 = pltpu.async_copy(mad_hbm.at[idx_v], madg_v, sem2)
    c1.wait()
    c2.wait()
    for i in range(_PER_W // _LANES):
        sl = pl.ds(i * _LANES, _LANES)
        o_v[sl] = (s_v[sl] - medg_v[sl]) / madg_v[sl]
    pltpu.sync_copy(o_v, out_hbm.at[pl.ds(base, _PER_W)])


def kernel(pred_rotmats, scores, grid_rotmats, medians, mads):
    pred9 = pred_rotmats.reshape(_B, 9)
    grid9 = grid_rotmats.reshape(_N_SO3, 9)
    cone_idx = _tc_cone_indices(pred9, grid9)
    return _sc_normalize(cone_idx, scores, medians, mads)


# speedup vs baseline: 1.9019x; 1.1645x over previous
import functools

import jax
import jax.numpy as jnp
from jax import lax
from jax.experimental import pallas as pl
from jax.experimental.pallas import tpu as pltpu
from jax.experimental.pallas import tpu_sc as plsc

_B = 4096
_N_SO3 = 4608
_N_PSI = 24
_N_CONES = 192

_ROW_BLK = 512
_N_ROW_BLKS = _B // _ROW_BLK

_NC = 2
_NS = 16
_NW = _NC * _NS
_PER_W = _B // _NW
_LANES = 16


def _tc_cone_body(grid_ref, pred_ref, out_ref):
    sim_t = lax.dot_general(
        grid_ref[...], pred_ref[...],
        dimension_numbers=(((1,), (1,)), ((), ())),
        preferred_element_type=jnp.float32,
    )
    cmax = jnp.max(sim_t.reshape(_N_CONES, _N_PSI, _ROW_BLK), axis=1)
    m = jnp.max(cmax, axis=0, keepdims=True)
    row = lax.broadcasted_iota(jnp.int32, (_N_CONES, _ROW_BLK), 0)
    out_ref[...] = jnp.min(jnp.where(cmax == m, row, _N_CONES), axis=0)


def _tc_cone_indices(pred9, grid9):
    return pl.pallas_call(
        _tc_cone_body,
        grid=(_N_ROW_BLKS,),
        in_specs=[
            pl.BlockSpec((_N_SO3, 9), lambda i: (0, 0)),
            pl.BlockSpec((_ROW_BLK, 9), lambda i: (i, 0)),
        ],
        out_specs=pl.BlockSpec((_ROW_BLK,), lambda i: (i,)),
        out_shape=jax.ShapeDtypeStruct((_B,), jnp.int32),
    )(grid9, pred9)


def _sc_normalize_body(cone_hbm, scores_hbm, med_hbm, mad_hbm, out_hbm,
                       idx_v, s_v, medg_v, madg_v, o_v, sem1, sem2):
    wid = lax.axis_index("s") * _NC + lax.axis_index("c")
    base = wid * _PER_W
    pltpu.sync_copy(cone_hbm.at[pl.ds(base, _PER_W)], idx_v)
    pltpu.sync_copy(scores_hbm.at[pl.ds(base, _PER_W)], s_v)
    c1 = pltpu.async_copy(med_hbm.at[idx_v], medg_v, sem1)
    c2 = pltpu.async_copy(mad_hbm.at[idx_v], madg_v, sem2)
    c1.wait()
    c2.wait()
    for i in range(_PER_W // _LANES):
        sl = pl.ds(i * _LANES, _LANES)
        o_v[sl] = (s_v[sl] - medg_v[sl]) / madg_v[sl]
    pltpu.sync_copy(o_v, out_hbm.at[pl.ds(base, _PER_W)])


@functools.cache
def _sc_normalize():
    mesh = plsc.VectorSubcoreMesh(core_axis_name="c", subcore_axis_name="s")
    return pl.kernel(
        _sc_normalize_body,
        mesh=mesh,
        out_type=jax.ShapeDtypeStruct((_B,), jnp.float32),
        scratch_types=[
            pltpu.VMEM((_PER_W,), jnp.int32),
            pltpu.VMEM((_PER_W,), jnp.float32),
            pltpu.VMEM((_PER_W,), jnp.float32),
            pltpu.VMEM((_PER_W,), jnp.float32),
            pltpu.VMEM((_PER_W,), jnp.float32),
            pltpu.SemaphoreType.DMA,
            pltpu.SemaphoreType.DMA,
        ],
    )


def kernel(pred_rotmats, scores, grid_rotmats, medians, mads):
    pred9 = pred_rotmats.reshape(_B, 9)
    grid9 = grid_rotmats.reshape(_N_SO3, 9)
    cone_idx = _tc_cone_indices(pred9, grid9)
    return _sc_normalize()(cone_idx, scores, medians, mads)


# speedup vs baseline: 1.9308x; 1.0152x over previous
import functools

import jax
import jax.numpy as jnp
from jax import lax
from jax.experimental import pallas as pl
from jax.experimental.pallas import tpu as pltpu
from jax.experimental.pallas import tpu_sc as plsc

_B = 4096
_N_SO3 = 4608
_N_PSI = 24
_N_CONES = 192

_ROW_BLK = 1024
_N_ROW_BLKS = _B // _ROW_BLK

_NC = 2
_NS = 16
_NW = _NC * _NS
_PER_W = _B // _NW
_LANES = 16


def _tc_cone_body(grid_ref, pred_ref, out_ref):
    sim_t = lax.dot_general(
        grid_ref[...], pred_ref[...],
        dimension_numbers=(((1,), (1,)), ((), ())),
        preferred_element_type=jnp.float32,
    )
    cmax = jnp.max(sim_t.reshape(_N_CONES, _N_PSI, _ROW_BLK), axis=1)
    m = jnp.max(cmax, axis=0, keepdims=True)
    row = lax.broadcasted_iota(jnp.int32, (_N_CONES, _ROW_BLK), 0)
    out_ref[...] = jnp.min(jnp.where(cmax == m, row, _N_CONES), axis=0)


def _tc_cone_indices(pred9, grid9):
    return pl.pallas_call(
        _tc_cone_body,
        grid=(_N_ROW_BLKS,),
        in_specs=[
            pl.BlockSpec((_N_SO3, 9), lambda i: (0, 0)),
            pl.BlockSpec((_ROW_BLK, 9), lambda i: (i, 0)),
        ],
        out_specs=pl.BlockSpec((_ROW_BLK,), lambda i: (i,)),
        out_shape=jax.ShapeDtypeStruct((_B,), jnp.int32),
    )(grid9, pred9)


def _sc_normalize_body(cone_hbm, scores_hbm, med_hbm, mad_hbm, out_hbm,
                       idx_v, s_v, medg_v, madg_v, o_v, sem1, sem2):
    wid = lax.axis_index("s") * _NC + lax.axis_index("c")
    base = wid * _PER_W
    pltpu.sync_copy(cone_hbm.at[pl.ds(base, _PER_W)], idx_v)
    pltpu.sync_copy(scores_hbm.at[pl.ds(base, _PER_W)], s_v)
    c1 = pltpu.async_copy(med_hbm.at[idx_v], medg_v, sem1)
    c2 = pltpu.async_copy(mad_hbm.at[idx_v], madg_v, sem2)
    c1.wait()
    c2.wait()
    for i in range(_PER_W // _LANES):
        sl = pl.ds(i * _LANES, _LANES)
        o_v[sl] = (s_v[sl] - medg_v[sl]) / madg_v[sl]
    pltpu.sync_copy(o_v, out_hbm.at[pl.ds(base, _PER_W)])


@functools.cache
def _sc_normalize():
    mesh = plsc.VectorSubcoreMesh(core_axis_name="c", subcore_axis_name="s")
    return pl.kernel(
        _sc_normalize_body,
        mesh=mesh,
        out_type=jax.ShapeDtypeStruct((_B,), jnp.float32),
        scratch_types=[
            pltpu.VMEM((_PER_W,), jnp.int32),
            pltpu.VMEM((_PER_W,), jnp.float32),
            pltpu.VMEM((_PER_W,), jnp.float32),
            pltpu.VMEM((_PER_W,), jnp.float32),
            pltpu.VMEM((_PER_W,), jnp.float32),
            pltpu.SemaphoreType.DMA,
            pltpu.SemaphoreType.DMA,
        ],
    )


def kernel(pred_rotmats, scores, grid_rotmats, medians, mads):
    pred9 = pred_rotmats.reshape(_B, 9)
    grid9 = grid_rotmats.reshape(_N_SO3, 9)
    cone_idx = _tc_cone_indices(pred9, grid9)
    return _sc_normalize()(cone_idx, scores, medians, mads)


# speedup vs baseline: 1.9481x; 1.0090x over previous
import functools

import jax
import jax.numpy as jnp
from jax import lax
from jax.experimental import pallas as pl
from jax.experimental.pallas import tpu as pltpu
from jax.experimental.pallas import tpu_sc as plsc

_B = 4096
_N_SO3 = 4608
_N_PSI = 24
_N_CONES = 192

_ROW_BLK = 1024
_N_ROW_BLKS = _B // _ROW_BLK

_NC = 2
_NS = 16
_NW = _NC * _NS
_PER_W = _B // _NW
_LANES = 16


def _tc_cone_body(grid_ref, pred_ref, out_ref):
    sim_t = lax.dot_general(
        grid_ref[...], pred_ref[...],
        dimension_numbers=(((1,), (1,)), ((), ())),
        preferred_element_type=jnp.float32,
    )
    cmax = jnp.max(sim_t.reshape(_N_CONES, _N_PSI, _ROW_BLK), axis=1)
    m = jnp.max(cmax, axis=0, keepdims=True)
    row = lax.broadcasted_iota(jnp.int32, (_N_CONES, _ROW_BLK), 0)
    out_ref[...] = jnp.min(jnp.where(cmax == m, row, _N_CONES), axis=0)


def _tc_cone_indices(pred9, grid9):
    return pl.pallas_call(
        _tc_cone_body,
        grid=(_N_ROW_BLKS,),
        in_specs=[
            pl.BlockSpec((_N_SO3, 9), lambda i: (0, 0)),
            pl.BlockSpec((_ROW_BLK, 9), lambda i: (i, 0)),
        ],
        out_specs=pl.BlockSpec((_ROW_BLK,), lambda i: (i,)),
        out_shape=jax.ShapeDtypeStruct((_B,), jnp.int32),
    )(grid9, pred9)


def _sc_normalize_body(cone_hbm, scores_hbm, med_hbm, mad_hbm, out_hbm,
                       idx_v, s_v, medg_v, madg_v, o_v, sem1, sem2):
    wid = lax.axis_index("s") * _NC + lax.axis_index("c")
    base = wid * _PER_W
    c_idx = pltpu.async_copy(cone_hbm.at[pl.ds(base, _PER_W)], idx_v, sem1)
    c_s = pltpu.async_copy(scores_hbm.at[pl.ds(base, _PER_W)], s_v, sem2)
    c_idx.wait()
    c1 = pltpu.async_copy(med_hbm.at[idx_v], medg_v, sem1)
    c2 = pltpu.async_copy(mad_hbm.at[idx_v], madg_v, sem2)
    c_s.wait()
    c1.wait()
    c2.wait()
    for i in range(_PER_W // _LANES):
        sl = pl.ds(i * _LANES, _LANES)
        o_v[sl] = (s_v[sl] - medg_v[sl]) / madg_v[sl]
    pltpu.sync_copy(o_v, out_hbm.at[pl.ds(base, _PER_W)])


@functools.cache
def _sc_normalize():
    mesh = plsc.VectorSubcoreMesh(core_axis_name="c", subcore_axis_name="s")
    return pl.kernel(
        _sc_normalize_body,
        mesh=mesh,
        out_type=jax.ShapeDtypeStruct((_B,), jnp.float32),
        scratch_types=[
            pltpu.VMEM((_PER_W,), jnp.int32),
            pltpu.VMEM((_PER_W,), jnp.float32),
            pltpu.VMEM((_PER_W,), jnp.float32),
            pltpu.VMEM((_PER_W,), jnp.float32),
            pltpu.VMEM((_PER_W,), jnp.float32),
            pltpu.SemaphoreType.DMA,
            pltpu.SemaphoreType.DMA,
        ],
    )


def kernel(pred_rotmats, scores, grid_rotmats, medians, mads):
    pred9 = pred_rotmats.reshape(_B, 9)
    grid9 = grid_rotmats.reshape(_N_SO3, 9)
    cone_idx = _tc_cone_indices(pred9, grid9)
    return _sc_normalize()(cone_idx, scores, medians, mads)


# speedup vs baseline: 1.9901x; 1.0216x over previous
import functools

import jax
import jax.numpy as jnp
from jax import lax
from jax.experimental import pallas as pl
from jax.experimental.pallas import tpu as pltpu
from jax.experimental.pallas import tpu_sc as plsc

_B = 4096
_N_SO3 = 4608
_N_PSI = 24
_N_CONES = 192

_ROW_BLK = 1024
_N_ROW_BLKS = _B // _ROW_BLK

_NC = 1
_NS = 16
_NW = _NC * _NS
_PER_W = _B // _NW
_LANES = 16


def _tc_cone_body(grid_ref, pred_ref, out_ref):
    sim_t = lax.dot_general(
        grid_ref[...], pred_ref[...],
        dimension_numbers=(((1,), (1,)), ((), ())),
        preferred_element_type=jnp.float32,
    )
    cmax = jnp.max(sim_t.reshape(_N_CONES, _N_PSI, _ROW_BLK), axis=1)
    m = jnp.max(cmax, axis=0, keepdims=True)
    row = lax.broadcasted_iota(jnp.int32, (_N_CONES, _ROW_BLK), 0)
    out_ref[...] = jnp.min(jnp.where(cmax == m, row, _N_CONES), axis=0)


def _tc_cone_indices(pred9, grid9):
    return pl.pallas_call(
        _tc_cone_body,
        grid=(_N_ROW_BLKS,),
        in_specs=[
            pl.BlockSpec((_N_SO3, 9), lambda i: (0, 0)),
            pl.BlockSpec((_ROW_BLK, 9), lambda i: (i, 0)),
        ],
        out_specs=pl.BlockSpec((_ROW_BLK,), lambda i: (i,)),
        out_shape=jax.ShapeDtypeStruct((_B,), jnp.int32),
    )(grid9, pred9)


def _sc_normalize_body(cone_hbm, scores_hbm, med_hbm, mad_hbm, out_hbm,
                       idx_v, s_v, medg_v, madg_v, o_v, sem1, sem2):
    wid = lax.axis_index("s") * _NC + lax.axis_index("c")
    base = wid * _PER_W
    c_idx = pltpu.async_copy(cone_hbm.at[pl.ds(base, _PER_W)], idx_v, sem1)
    c_s = pltpu.async_copy(scores_hbm.at[pl.ds(base, _PER_W)], s_v, sem2)
    c_idx.wait()
    c1 = pltpu.async_copy(med_hbm.at[idx_v], medg_v, sem1)
    c2 = pltpu.async_copy(mad_hbm.at[idx_v], madg_v, sem2)
    c_s.wait()
    c1.wait()
    c2.wait()
    for i in range(_PER_W // _LANES):
        sl = pl.ds(i * _LANES, _LANES)
        o_v[sl] = (s_v[sl] - medg_v[sl]) / madg_v[sl]
    pltpu.sync_copy(o_v, out_hbm.at[pl.ds(base, _PER_W)])


@functools.cache
def _sc_normalize():
    mesh = plsc.VectorSubcoreMesh(
        core_axis_name="c", subcore_axis_name="s", num_cores=_NC)
    return pl.kernel(
        _sc_normalize_body,
        mesh=mesh,
        out_type=jax.ShapeDtypeStruct((_B,), jnp.float32),
        scratch_types=[
            pltpu.VMEM((_PER_W,), jnp.int32),
            pltpu.VMEM((_PER_W,), jnp.float32),
            pltpu.VMEM((_PER_W,), jnp.float32),
            pltpu.VMEM((_PER_W,), jnp.float32),
            pltpu.VMEM((_PER_W,), jnp.float32),
            pltpu.SemaphoreType.DMA,
            pltpu.SemaphoreType.DMA,
        ],
    )


def kernel(pred_rotmats, scores, grid_rotmats, medians, mads):
    pred9 = pred_rotmats.reshape(_B, 9)
    grid9 = grid_rotmats.reshape(_N_SO3, 9)
    cone_idx = _tc_cone_indices(pred9, grid9)
    return _sc_normalize()(cone_idx, scores, medians, mads)


# speedup vs baseline: 2.0039x; 1.0069x over previous
import functools

import jax
import jax.numpy as jnp
from jax import lax
from jax.experimental import pallas as pl
from jax.experimental.pallas import tpu as pltpu
from jax.experimental.pallas import tpu_sc as plsc

_B = 4096
_N_SO3 = 4608
_N_PSI = 24
_N_CONES = 192

_ROW_BLK = 1024
_BH = _B // 2

_NC = 1
_NS = 16
_NW = _NC * _NS
_PER_W = _BH // _NW
_LANES = 16


def _tc_cone_body(grid_ref, pred_ref, out_ref):
    sim_t = lax.dot_general(
        grid_ref[...], pred_ref[...],
        dimension_numbers=(((1,), (1,)), ((), ())),
        preferred_element_type=jnp.float32,
    )
    cmax = jnp.max(sim_t.reshape(_N_CONES, _N_PSI, _ROW_BLK), axis=1)
    m = jnp.max(cmax, axis=0, keepdims=True)
    row = lax.broadcasted_iota(jnp.int32, (_N_CONES, _ROW_BLK), 0)
    out_ref[...] = jnp.min(jnp.where(cmax == m, row, _N_CONES), axis=0)


def _tc_cone_indices(pred9, grid9):
    return pl.pallas_call(
        _tc_cone_body,
        grid=(_BH // _ROW_BLK,),
        in_specs=[
            pl.BlockSpec((_N_SO3, 9), lambda i: (0, 0)),
            pl.BlockSpec((_ROW_BLK, 9), lambda i: (i, 0)),
        ],
        out_specs=pl.BlockSpec((_ROW_BLK,), lambda i: (i,)),
        out_shape=jax.ShapeDtypeStruct((_BH,), jnp.int32),
    )(grid9, pred9)


def _sc_normalize_body(cone_hbm, scores_hbm, med_hbm, mad_hbm, out_hbm,
                       idx_v, s_v, medg_v, madg_v, o_v, sem1, sem2):
    wid = lax.axis_index("s") * _NC + lax.axis_index("c")
    base = wid * _PER_W
    c_idx = pltpu.async_copy(cone_hbm.at[pl.ds(base, _PER_W)], idx_v, sem1)
    c_s = pltpu.async_copy(scores_hbm.at[pl.ds(base, _PER_W)], s_v, sem2)
    c_idx.wait()
    c1 = pltpu.async_copy(med_hbm.at[idx_v], medg_v, sem1)
    c2 = pltpu.async_copy(mad_hbm.at[idx_v], madg_v, sem2)
    c_s.wait()
    c1.wait()
    c2.wait()
    for i in range(_PER_W // _LANES):
        sl = pl.ds(i * _LANES, _LANES)
        o_v[sl] = (s_v[sl] - medg_v[sl]) / madg_v[sl]
    pltpu.sync_copy(o_v, out_hbm.at[pl.ds(base, _PER_W)])


@functools.cache
def _sc_normalize():
    mesh = plsc.VectorSubcoreMesh(
        core_axis_name="c", subcore_axis_name="s", num_cores=_NC)
    return pl.kernel(
        _sc_normalize_body,
        mesh=mesh,
        out_type=jax.ShapeDtypeStruct((_BH,), jnp.float32),
        scratch_types=[
            pltpu.VMEM((_PER_W,), jnp.int32),
            pltpu.VMEM((_PER_W,), jnp.float32),
            pltpu.VMEM((_PER_W,), jnp.float32),
            pltpu.VMEM((_PER_W,), jnp.float32),
            pltpu.VMEM((_PER_W,), jnp.float32),
            pltpu.SemaphoreType.DMA,
            pltpu.SemaphoreType.DMA,
        ],
    )


def kernel(pred_rotmats, scores, grid_rotmats, medians, mads):
    pred9 = pred_rotmats.reshape(_B, 9)
    grid9 = grid_rotmats.reshape(_N_SO3, 9)
    sc = _sc_normalize()
    cone1 = _tc_cone_indices(pred9[:_BH], grid9)
    cone2 = _tc_cone_indices(pred9[_BH:], grid9)
    out1 = sc(cone1, scores[:_BH], medians, mads)
    out2 = sc(cone2, scores[_BH:], medians, mads)
    return jnp.concatenate([out1, out2])
